# baseline (speedup 1.0000x reference)
import functools

import jax
import jax.numpy as jnp
from jax import lax
from jax.experimental import pallas as pl
from jax.experimental.pallas import tpu as pltpu
from jax.experimental.pallas import tpu_sc as plsc

ALPHA = 0.3
_BS = 1024
_L = 16
_NT = 16


def _copy_body(hid_ref, out_ref):
    out_ref[...] = hid_ref[...]


def _tc_copy(flat):
    rows, d = flat.shape
    return pl.pallas_call(
        _copy_body,
        grid=(rows // _BS,),
        in_specs=[pl.BlockSpec((_BS, d), lambda i: (i, 0))],
        out_specs=pl.BlockSpec((_BS, d), lambda i: (i, 0)),
        out_shape=jax.ShapeDtypeStruct((rows, d), flat.dtype),
    )(flat)


def _sc_body(hid_ref, attr_ref, out_ref, h_v, attr_v, row_v, stage_v, shared_v,
             gather_idx_v, win_v, sem):
    rows, d = hid_ref.shape
    k = attr_ref.shape[0]
    nchunk = d // _L
    per_tile = k // _NT
    half = per_tile // 2

    wid = lax.axis_index("s")

    pltpu.sync_copy(hid_ref.at[rows // 2 - 1], h_v.at[0])
    pltpu.sync_copy(hid_ref.at[rows - 1], h_v.at[1])

    def score_half(hf, carry):
        base = wid * per_tile + hf * half
        pltpu.sync_copy(attr_ref.at[pl.ds(base, half)], attr_v)

        def one_attr(a, c):
            bs0, bi0, bs1, bi1 = c

            z = jnp.zeros((_L,), jnp.float32)
            acc0 = [z, z]
            acc1 = [z, z]
            for cc in range(nchunk):
                av = attr_v[a, pl.ds(cc * _L, _L)]
                acc0[cc % 2] = acc0[cc % 2] + av * h_v[0, pl.ds(cc * _L, _L)]
                acc1[cc % 2] = acc1[cc % 2] + av * h_v[1, pl.ds(cc * _L, _L)]
            s0 = plsc.cumsum(acc0[0] + acc0[1])[_L - 1]
            s1 = plsc.cumsum(acc1[0] + acc1[1])[_L - 1]
            gidx = base + a
            better0 = s0 > bs0
            better1 = s1 > bs1
            bs0 = jnp.where(better0, s0, bs0)
            bi0 = jnp.where(better0, gidx, bi0)
            bs1 = jnp.where(better1, s1, bs1)
            bi1 = jnp.where(better1, gidx, bi1)
            return bs0, bi0, bs1, bi1

        return lax.fori_loop(0, half, one_attr, carry)

    neg = jnp.float32(-3e38)
    best = (neg, jnp.int32(0), neg, jnp.int32(0))
    best = score_half(0, best)
    bs0, bi0, bs1, bi1 = score_half(1, best)

    lane = lax.iota(jnp.int32, _L)
    pub = jnp.where(lane == 0, bs0,
                    jnp.where(lane == 1, bs1,
                              jnp.where(lane == 2, bi0.astype(jnp.float32),
                                        jnp.where(lane == 3,
                                                  bi1.astype(jnp.float32),
                                                  0.0))))
    stage_v[...] = pub
    pltpu.sync_copy(stage_v, shared_v.at[wid])
    plsc.subcore_barrier()
    pltpu.sync_copy(shared_v, win_v)

    gbs0, gbi0 = neg, jnp.int32(0)
    gbs1, gbi1 = neg, jnp.int32(0)
    for t in range(_NT):
        rv = win_v[t, :]
        v0 = rv[0]
        v1 = rv[1]
        i0 = rv[2].astype(jnp.int32)
        i1 = rv[3].astype(jnp.int32)
        b0 = v0 > gbs0
        b1 = v1 > gbs1
        gbs0 = jnp.where(b0, v0, gbs0)
        gbi0 = jnp.where(b0, i0, gbi0)
        gbs1 = jnp.where(b1, v1, gbs1)
        gbi1 = jnp.where(b1, i1, gbi1)

    def finalize(b, gidx):
        @pl.when(wid == gidx // per_tile)
        def _():
            gather_idx_v[...] = jnp.full((_L,), gidx, jnp.int32)
            pltpu.async_copy(attr_ref.at[gather_idx_v], row_v, sem).wait()
            pltpu.sync_copy(row_v.at[0], out_ref.at[b])

    finalize(0, gbi0)
    finalize(1, gbi1)


def _sc_rows(flat, attractors):
    d = flat.shape[1]
    k = attractors.shape[0]
    half = k // _NT // 2
    mesh = plsc.VectorSubcoreMesh(
        core_axis_name="c", subcore_axis_name="s", num_cores=1)
    f = pl.kernel(
        _sc_body,
        out_type=jax.ShapeDtypeStruct((2, d), jnp.float32),
        mesh=mesh,
        compiler_params=pltpu.CompilerParams(needs_layout_passes=False),
        scratch_types=[
            pltpu.VMEM((2, d), jnp.float32),
            pltpu.VMEM((half, d), jnp.float32),
            pltpu.VMEM((_L, d), jnp.float32),
            pltpu.VMEM((_L,), jnp.float32),
            pltpu.VMEM_SHARED((_NT, _L), jnp.float32),
            pltpu.VMEM((_L,), jnp.int32),
            pltpu.VMEM((_NT, _L), jnp.float32),
            pltpu.SemaphoreType.DMA,
        ],
    )
    return f(flat, attractors)


def _merge_body(cp_ref, rows_ref, out_ref):
    i = pl.program_id(0)
    out_ref[...] = cp_ref[...]
    h = cp_ref[7, :]
    norm = jnp.sqrt(jnp.sum(h * h))
    out_ref[7, :] = (1.0 - ALPHA) * h + (ALPHA * norm) * rows_ref[i, :]


def _merge(copied, new_rows):
    rows, d = copied.shape
    nblk = rows // 8
    return pl.pallas_call(
        _merge_body,
        grid=(2,),
        in_specs=[
            pl.BlockSpec((8, d), lambda i: ((i + 1) * (nblk // 2) - 1, 0)),
            pl.BlockSpec((2, d), lambda i: (0, 0)),
        ],
        out_specs=pl.BlockSpec((8, d), lambda i: ((i + 1) * (nblk // 2) - 1, 0)),
        out_shape=jax.ShapeDtypeStruct((rows, d), copied.dtype),
        input_output_aliases={0: 0},
    )(copied, new_rows)


def kernel(hidden_states, attractors):
    b, s, d = hidden_states.shape
    flat = hidden_states.reshape(b * s, d)
    copied = _tc_copy(flat)
    new_rows = _sc_rows(flat, attractors)
    out = _merge(copied, new_rows)
    return out.reshape(b, s, d)


# speedup vs baseline: 1.1002x; 1.1002x over previous
import functools

import jax
import jax.numpy as jnp
from jax import lax
from jax.experimental import pallas as pl
from jax.experimental.pallas import tpu as pltpu
from jax.experimental.pallas import tpu_sc as plsc

ALPHA = 0.3
_BS = 1024
_L = 16
_NT = 16


def _copy_body(hid_ref, out_ref):
    out_ref[...] = hid_ref[...]


def _tc_copy(flat):
    rows, d = flat.shape
    return pl.pallas_call(
        _copy_body,
        grid=(rows // _BS,),
        in_specs=[pl.BlockSpec((_BS, d), lambda i: (i, 0))],
        out_specs=pl.BlockSpec((_BS, d), lambda i: (i, 0)),
        out_shape=jax.ShapeDtypeStruct((rows, d), flat.dtype),
    )(flat)


def _sc_body(hid_ref, attr_ref, out_ref, h_v, attr_v, row_v, stage_v, shared_v,
             gather_idx_v, win_v, sem):
    rows, d = hid_ref.shape
    k = attr_ref.shape[0]
    nchunk = d // _L
    per_tile = k // _NT
    half = per_tile // 2

    wid = lax.axis_index("s")

    pltpu.sync_copy(hid_ref.at[rows // 2 - 1], h_v.at[0])
    pltpu.sync_copy(hid_ref.at[rows - 1], h_v.at[1])

    def score_half(hf, carry):
        base = wid * per_tile + hf * half
        pltpu.sync_copy(attr_ref.at[pl.ds(base, half)], attr_v)

        def one_attr(a, c):
            bs0, bi0, bs1, bi1 = c

            z = jnp.zeros((_L,), jnp.float32)

            def dot_chunks(step, accs):
                a00, a01, a10, a11 = accs
                for u in range(4):
                    off = (step * 4 + u) * _L
                    av = attr_v[a, pl.ds(off, _L)]
                    if u % 2 == 0:
                        a00 = a00 + av * h_v[0, pl.ds(off, _L)]
                        a10 = a10 + av * h_v[1, pl.ds(off, _L)]
                    else:
                        a01 = a01 + av * h_v[0, pl.ds(off, _L)]
                        a11 = a11 + av * h_v[1, pl.ds(off, _L)]
                return a00, a01, a10, a11

            a00, a01, a10, a11 = lax.fori_loop(
                0, nchunk // 4, dot_chunks, (z, z, z, z))
            s0 = plsc.cumsum(a00 + a01)[_L - 1]
            s1 = plsc.cumsum(a10 + a11)[_L - 1]
            gidx = base + a
            better0 = s0 > bs0
            better1 = s1 > bs1
            bs0 = jnp.where(better0, s0, bs0)
            bi0 = jnp.where(better0, gidx, bi0)
            bs1 = jnp.where(better1, s1, bs1)
            bi1 = jnp.where(better1, gidx, bi1)
            return bs0, bi0, bs1, bi1

        return lax.fori_loop(0, half, one_attr, carry)

    neg = jnp.float32(-3e38)
    best = (neg, jnp.int32(0), neg, jnp.int32(0))
    best = score_half(0, best)
    bs0, bi0, bs1, bi1 = score_half(1, best)

    lane = lax.iota(jnp.int32, _L)
    pub = jnp.where(lane == 0, bs0,
                    jnp.where(lane == 1, bs1,
                              jnp.where(lane == 2, bi0.astype(jnp.float32),
                                        jnp.where(lane == 3,
                                                  bi1.astype(jnp.float32),
                                                  0.0))))
    stage_v[...] = pub
    pltpu.sync_copy(stage_v, shared_v.at[wid])
    plsc.subcore_barrier()
    pltpu.sync_copy(shared_v, win_v)

    gbs0, gbi0 = neg, jnp.int32(0)
    gbs1, gbi1 = neg, jnp.int32(0)
    for t in range(_NT):
        rv = win_v[t, :]
        v0 = rv[0]
        v1 = rv[1]
        i0 = rv[2].astype(jnp.int32)
        i1 = rv[3].astype(jnp.int32)
        b0 = v0 > gbs0
        b1 = v1 > gbs1
        gbs0 = jnp.where(b0, v0, gbs0)
        gbi0 = jnp.where(b0, i0, gbi0)
        gbs1 = jnp.where(b1, v1, gbs1)
        gbi1 = jnp.where(b1, i1, gbi1)

    def finalize(b, gidx):
        @pl.when(wid == gidx // per_tile)
        def _():
            gather_idx_v[...] = jnp.full((_L,), gidx, jnp.int32)
            pltpu.async_copy(attr_ref.at[gather_idx_v], row_v, sem).wait()
            pltpu.sync_copy(row_v.at[0], out_ref.at[b])

    finalize(0, gbi0)
    finalize(1, gbi1)


def _sc_rows(flat, attractors):
    d = flat.shape[1]
    k = attractors.shape[0]
    half = k // _NT // 2
    mesh = plsc.VectorSubcoreMesh(
        core_axis_name="c", subcore_axis_name="s", num_cores=1)
    f = pl.kernel(
        _sc_body,
        out_type=jax.ShapeDtypeStruct((2, d), jnp.float32),
        mesh=mesh,
        compiler_params=pltpu.CompilerParams(needs_layout_passes=False),
        scratch_types=[
            pltpu.VMEM((2, d), jnp.float32),
            pltpu.VMEM((half, d), jnp.float32),
            pltpu.VMEM((_L, d), jnp.float32),
            pltpu.VMEM((_L,), jnp.float32),
            pltpu.VMEM_SHARED((_NT, _L), jnp.float32),
            pltpu.VMEM((_L,), jnp.int32),
            pltpu.VMEM((_NT, _L), jnp.float32),
            pltpu.SemaphoreType.DMA,
        ],
    )
    return f(flat, attractors)


def _merge_body(cp_ref, rows_ref, out_ref):
    i = pl.program_id(0)
    out_ref[...] = cp_ref[...]
    h = cp_ref[7, :]
    norm = jnp.sqrt(jnp.sum(h * h))
    out_ref[7, :] = (1.0 - ALPHA) * h + (ALPHA * norm) * rows_ref[i, :]


def _merge(copied, new_rows):
    rows, d = copied.shape
    nblk = rows // 8
    return pl.pallas_call(
        _merge_body,
        grid=(2,),
        in_specs=[
            pl.BlockSpec((8, d), lambda i: ((i + 1) * (nblk // 2) - 1, 0)),
            pl.BlockSpec((2, d), lambda i: (0, 0)),
        ],
        out_specs=pl.BlockSpec((8, d), lambda i: ((i + 1) * (nblk // 2) - 1, 0)),
        out_shape=jax.ShapeDtypeStruct((rows, d), copied.dtype),
        input_output_aliases={0: 0},
    )(copied, new_rows)


def kernel(hidden_states, attractors):
    b, s, d = hidden_states.shape
    flat = hidden_states.reshape(b * s, d)
    copied = _tc_copy(flat)
    new_rows = _sc_rows(flat, attractors)
    out = _merge(copied, new_rows)
    return out.reshape(b, s, d)


# speedup vs baseline: 1.3206x; 1.2004x over previous
import jax
import jax.numpy as jnp
from jax import lax
from jax.experimental import pallas as pl
from jax.experimental.pallas import tpu as pltpu

ALPHA = 0.3
_CH = 512
_NBUF = 4
_LAG = 2
_TAIL = 8


def _compute_rows(tails_ref, attr_ref):
    b = tails_ref.shape[0]
    h = tails_ref[:, _TAIL - 1, :]
    norm = jnp.sqrt(jnp.sum(h * h, axis=1, keepdims=True))
    safe = jnp.maximum(norm, 1e-12)
    h_n = h / safe
    attr = attr_ref[...]
    sims = lax.dot_general(h_n, attr, (((1,), (1,)), ((), ())),
                           preferred_element_type=jnp.float32)
    k = sims.shape[1]
    iota = lax.broadcasted_iota(jnp.int32, (b, k), 1)
    m = jnp.max(sims, axis=1, keepdims=True)
    idx = jnp.min(jnp.where(sims == m, iota, k), axis=1, keepdims=True)
    one_hot = (iota == idx).astype(jnp.float32)
    nearest = lax.dot_general(one_hot, attr, (((1,), (0,)), ((), ())),
                              preferred_element_type=jnp.float32)
    tails_ref[:, _TAIL - 1, :] = h + ALPHA * (nearest - h_n) * norm


def _body(hid_ref, attr_hbm, out_ref, buf_ref, attr_vmem, tails_vmem,
          sem_in, sem_out, sem_attr, sem_tail):
    rows, d = hid_ref.shape
    half = rows // 2

    chunks = []
    for base, stop in ((0, half - _TAIL), (half, rows - _TAIL)):
        lo = base
        while lo < stop:
            sz = min(_CH, stop - lo)
            chunks.append((lo, sz))
            lo += sz
    nc = len(chunks)

    attr_cp = pltpu.make_async_copy(attr_hbm, attr_vmem, sem_attr)
    attr_cp.start()
    tail_cps = []
    for i in range(2):
        cp = pltpu.make_async_copy(
            hid_ref.at[pl.ds((i + 1) * half - _TAIL, _TAIL), :],
            tails_vmem.at[i], sem_tail.at[i])
        cp.start()
        tail_cps.append(cp)

    def in_cp(c):
        lo, sz = chunks[c]
        return pltpu.make_async_copy(
            hid_ref.at[pl.ds(lo, sz), :], buf_ref.at[c % _NBUF, pl.ds(0, sz)],
            sem_in.at[c % _NBUF])

    def out_cp(c):
        lo, sz = chunks[c]
        return pltpu.make_async_copy(
            buf_ref.at[c % _NBUF, pl.ds(0, sz)], out_ref.at[pl.ds(lo, sz), :],
            sem_out.at[c % _NBUF])

    for step in range(nc + _LAG):
        if step < nc:
            if step >= _NBUF:
                out_cp(step - _NBUF).wait()
            in_cp(step).start()
        c_proc = step - _LAG
        if c_proc >= 0:
            in_cp(c_proc).wait()
            out_cp(c_proc).start()

    attr_cp.wait()
    for cp in tail_cps:
        cp.wait()
    _compute_rows(tails_vmem, attr_vmem)

    tail_out = []
    for i in range(2):
        cp = pltpu.make_async_copy(
            tails_vmem.at[i],
            out_ref.at[pl.ds((i + 1) * half - _TAIL, _TAIL), :],
            sem_tail.at[i])
        cp.start()
        tail_out.append(cp)

    for c in range(nc - _NBUF, nc):
        out_cp(c).wait()
    for cp in tail_out:
        cp.wait()


def kernel(hidden_states, attractors):
    b, s, d = hidden_states.shape
    k = attractors.shape[0]
    flat = hidden_states.reshape(b * s, d)
    out = pl.pallas_call(
        _body,
        in_specs=[
            pl.BlockSpec(memory_space=pltpu.HBM),
            pl.BlockSpec(memory_space=pltpu.HBM),
        ],
        out_specs=pl.BlockSpec(memory_space=pltpu.HBM),
        out_shape=jax.ShapeDtypeStruct((b * s, d), hidden_states.dtype),
        scratch_shapes=[
            pltpu.VMEM((_NBUF, _CH, d), jnp.float32),
            pltpu.VMEM((k, d), jnp.float32),
            pltpu.VMEM((2, _TAIL, d), jnp.float32),
            pltpu.SemaphoreType.DMA((_NBUF,)),
            pltpu.SemaphoreType.DMA((_NBUF,)),
            pltpu.SemaphoreType.DMA,
            pltpu.SemaphoreType.DMA((2,)),
        ],
    )(flat, attractors)
    return out.reshape(b, s, d)


# speedup vs baseline: 1.3318x; 1.0084x over previous
import jax
import jax.numpy as jnp
from jax import lax
from jax.experimental import pallas as pl
from jax.experimental.pallas import tpu as pltpu

ALPHA = 0.3
_CH = 1024
_NBUF = 3
_LAG = 2
_TAIL = 8


def _compute_rows(tails_ref, attr_ref):
    b = tails_ref.shape[0]
    h = tails_ref[:, _TAIL - 1, :]
    norm = jnp.sqrt(jnp.sum(h * h, axis=1, keepdims=True))
    safe = jnp.maximum(norm, 1e-12)
    h_n = h / safe
    attr = attr_ref[...]
    sims = lax.dot_general(h_n, attr, (((1,), (1,)), ((), ())),
                           preferred_element_type=jnp.float32)
    k = sims.shape[1]
    iota = lax.broadcasted_iota(jnp.int32, (b, k), 1)
    m = jnp.max(sims, axis=1, keepdims=True)
    idx = jnp.min(jnp.where(sims == m, iota, k), axis=1, keepdims=True)
    one_hot = (iota == idx).astype(jnp.float32)
    nearest = lax.dot_general(one_hot, attr, (((1,), (0,)), ((), ())),
                              preferred_element_type=jnp.float32)
    tails_ref[:, _TAIL - 1, :] = h + ALPHA * (nearest - h_n) * norm


def _body(hid_ref, attr_hbm, out_ref, buf_ref, attr_vmem, tails_vmem,
          sem_in, sem_out, sem_attr, sem_tail):
    rows, d = hid_ref.shape
    half = rows // 2

    chunks = []
    for base, stop in ((0, half - _TAIL), (half, rows - _TAIL)):
        lo = base
        while lo < stop:
            sz = min(_CH, stop - lo)
            chunks.append((lo, sz))
            lo += sz
    nc = len(chunks)

    attr_cp = pltpu.make_async_copy(attr_hbm, attr_vmem, sem_attr)
    attr_cp.start()
    tail_cps = []
    for i in range(2):
        cp = pltpu.make_async_copy(
            hid_ref.at[pl.ds((i + 1) * half - _TAIL, _TAIL), :],
            tails_vmem.at[i], sem_tail.at[i])
        cp.start()
        tail_cps.append(cp)

    def in_cp(c):
        lo, sz = chunks[c]
        return pltpu.make_async_copy(
            hid_ref.at[pl.ds(lo, sz), :], buf_ref.at[c % _NBUF, pl.ds(0, sz)],
            sem_in.at[c % _NBUF])

    def out_cp(c):
        lo, sz = chunks[c]
        return pltpu.make_async_copy(
            buf_ref.at[c % _NBUF, pl.ds(0, sz)], out_ref.at[pl.ds(lo, sz), :],
            sem_out.at[c % _NBUF])

    for step in range(nc + _LAG):
        if step < nc:
            if step >= _NBUF:
                out_cp(step - _NBUF).wait()
            in_cp(step).start()
        c_proc = step - _LAG
        if c_proc >= 0:
            in_cp(c_proc).wait()
            out_cp(c_proc).start()

    attr_cp.wait()
    for cp in tail_cps:
        cp.wait()
    _compute_rows(tails_vmem, attr_vmem)

    tail_out = []
    for i in range(2):
        cp = pltpu.make_async_copy(
            tails_vmem.at[i],
            out_ref.at[pl.ds((i + 1) * half - _TAIL, _TAIL), :],
            sem_tail.at[i])
        cp.start()
        tail_out.append(cp)

    for c in range(nc - _NBUF, nc):
        out_cp(c).wait()
    for cp in tail_out:
        cp.wait()


def kernel(hidden_states, attractors):
    b, s, d = hidden_states.shape
    k = attractors.shape[0]
    flat = hidden_states.reshape(b * s, d)
    out = pl.pallas_call(
        _body,
        in_specs=[
            pl.BlockSpec(memory_space=pltpu.HBM),
            pl.BlockSpec(memory_space=pltpu.HBM),
        ],
        out_specs=pl.BlockSpec(memory_space=pltpu.HBM),
        out_shape=jax.ShapeDtypeStruct((b * s, d), hidden_states.dtype),
        scratch_shapes=[
            pltpu.VMEM((_NBUF, _CH, d), jnp.float32),
            pltpu.VMEM((k, d), jnp.float32),
            pltpu.VMEM((2, _TAIL, d), jnp.float32),
            pltpu.SemaphoreType.DMA((_NBUF,)),
            pltpu.SemaphoreType.DMA((_NBUF,)),
            pltpu.SemaphoreType.DMA,
            pltpu.SemaphoreType.DMA((2,)),
        ],
    )(flat, attractors)
    return out.reshape(b, s, d)


# speedup vs baseline: 1.3322x; 1.0003x over previous
import jax
import jax.numpy as jnp
from jax import lax
from jax.experimental import pallas as pl
from jax.experimental.pallas import tpu as pltpu

ALPHA = 0.3
_CH = 1024
_NBUF = 4
_LAG = 2
_TAIL = 8


def _compute_rows(tails_ref, attr_ref):
    b = tails_ref.shape[0]
    h = tails_ref[:, _TAIL - 1, :]
    norm = jnp.sqrt(jnp.sum(h * h, axis=1, keepdims=True))
    safe = jnp.maximum(norm, 1e-12)
    h_n = h / safe
    attr = attr_ref[...]
    sims = lax.dot_general(h_n, attr, (((1,), (1,)), ((), ())),
                           preferred_element_type=jnp.float32)
    k = sims.shape[1]
    iota = lax.broadcasted_iota(jnp.int32, (b, k), 1)
    m = jnp.max(sims, axis=1, keepdims=True)
    idx = jnp.min(jnp.where(sims == m, iota, k), axis=1, keepdims=True)
    one_hot = (iota == idx).astype(jnp.float32)
    nearest = lax.dot_general(one_hot, attr, (((1,), (0,)), ((), ())),
                              preferred_element_type=jnp.float32)
    tails_ref[:, _TAIL - 1, :] = h + ALPHA * (nearest - h_n) * norm


def _body(hid_ref, attr_hbm, out_ref, buf_ref, attr_vmem, tails_vmem,
          sem_in, sem_out, sem_attr, sem_tail):
    rows, d = hid_ref.shape
    half = rows // 2

    chunks = []
    for base, stop in ((0, half - _TAIL), (half, rows - _TAIL)):
        lo = base
        while lo < stop:
            sz = min(_CH, stop - lo)
            chunks.append((lo, sz))
            lo += sz
    nc = len(chunks)

    attr_cp = pltpu.make_async_copy(attr_hbm, attr_vmem, sem_attr)
    attr_cp.start()
    tail_cps = []
    for i in range(2):
        cp = pltpu.make_async_copy(
            hid_ref.at[pl.ds((i + 1) * half - _TAIL, _TAIL), :],
            tails_vmem.at[i], sem_tail.at[i])
        cp.start()
        tail_cps.append(cp)

    def in_cp(c):
        lo, sz = chunks[c]
        return pltpu.make_async_copy(
            hid_ref.at[pl.ds(lo, sz), :], buf_ref.at[c % _NBUF, pl.ds(0, sz)],
            sem_in.at[c % _NBUF])

    def out_cp(c):
        lo, sz = chunks[c]
        return pltpu.make_async_copy(
            buf_ref.at[c % _NBUF, pl.ds(0, sz)], out_ref.at[pl.ds(lo, sz), :],
            sem_out.at[c % _NBUF])

    for step in range(nc + _LAG):
        if step < nc:
            if step >= _NBUF:
                out_cp(step - _NBUF).wait()
            in_cp(step).start()
        c_proc = step - _LAG
        if c_proc >= 0:
            in_cp(c_proc).wait()
            out_cp(c_proc).start()

    attr_cp.wait()
    for cp in tail_cps:
        cp.wait()
    _compute_rows(tails_vmem, attr_vmem)

    tail_out = []
    for i in range(2):
        cp = pltpu.make_async_copy(
            tails_vmem.at[i],
            out_ref.at[pl.ds((i + 1) * half - _TAIL, _TAIL), :],
            sem_tail.at[i])
        cp.start()
        tail_out.append(cp)

    for c in range(nc - _NBUF, nc):
        out_cp(c).wait()
    for cp in tail_out:
        cp.wait()


def kernel(hidden_states, attractors):
    b, s, d = hidden_states.shape
    k = attractors.shape[0]
    flat = hidden_states.reshape(b * s, d)
    out = pl.pallas_call(
        _body,
        in_specs=[
            pl.BlockSpec(memory_space=pltpu.HBM),
            pl.BlockSpec(memory_space=pltpu.HBM),
        ],
        out_specs=pl.BlockSpec(memory_space=pltpu.HBM),
        out_shape=jax.ShapeDtypeStruct((b * s, d), hidden_states.dtype),
        scratch_shapes=[
            pltpu.VMEM((_NBUF, _CH, d), jnp.float32),
            pltpu.VMEM((k, d), jnp.float32),
            pltpu.VMEM((2, _TAIL, d), jnp.float32),
            pltpu.SemaphoreType.DMA((_NBUF,)),
            pltpu.SemaphoreType.DMA((_NBUF,)),
            pltpu.SemaphoreType.DMA,
            pltpu.SemaphoreType.DMA((2,)),
        ],
    )(flat, attractors)
    return out.reshape(b, s, d)


# speedup vs baseline: 1.3346x; 1.0018x over previous
import jax
import jax.numpy as jnp
from jax import lax
from jax.experimental import pallas as pl
from jax.experimental.pallas import tpu as pltpu

ALPHA = 0.3
_CH = 1024
_NBUF = 3
_LAG = 2
_TAIL = 8


def _compute_rows(tails_ref, attr_ref):
    b = tails_ref.shape[0]
    h = tails_ref[:, _TAIL - 1, :]
    norm = jnp.sqrt(jnp.sum(h * h, axis=1, keepdims=True))
    safe = jnp.maximum(norm, 1e-12)
    h_n = h / safe
    attr = attr_ref[...]
    sims = lax.dot_general(h_n, attr, (((1,), (1,)), ((), ())),
                           preferred_element_type=jnp.float32)
    k = sims.shape[1]
    iota = lax.broadcasted_iota(jnp.int32, (b, k), 1)
    m = jnp.max(sims, axis=1, keepdims=True)
    idx = jnp.min(jnp.where(sims == m, iota, k), axis=1, keepdims=True)
    one_hot = (iota == idx).astype(jnp.float32)
    nearest = lax.dot_general(one_hot, attr, (((1,), (0,)), ((), ())),
                              preferred_element_type=jnp.float32)
    tails_ref[:, _TAIL - 1, :] = h + ALPHA * (nearest - h_n) * norm


def _body(hid_ref, attr_hbm, out_ref, buf_ref, attr_vmem, tails_vmem,
          sem_in, sem_out, sem_attr, sem_tail):
    rows, d = hid_ref.shape
    half = rows // 2

    chunks = []
    for base, stop in ((0, half - _TAIL), (half, rows - _TAIL)):
        lo = base
        while lo < stop:
            sz = min(_CH, stop - lo)
            chunks.append((lo, sz))
            lo += sz
    nc = len(chunks)

    attr_cp = pltpu.make_async_copy(attr_hbm, attr_vmem, sem_attr)
    attr_cp.start()
    tail_cps = []
    for i in range(2):
        cp = pltpu.make_async_copy(
            hid_ref.at[pl.ds((i + 1) * half - _TAIL, _TAIL), :],
            tails_vmem.at[i], sem_tail.at[i])
        cp.start()
        tail_cps.append(cp)

    def in_cp(c):
        lo, sz = chunks[c]
        return pltpu.make_async_copy(
            hid_ref.at[pl.ds(lo, sz), :], buf_ref.at[c % _NBUF, pl.ds(0, sz)],
            sem_in.at[c % _NBUF])

    def out_cp(c):
        lo, sz = chunks[c]
        return pltpu.make_async_copy(
            buf_ref.at[c % _NBUF, pl.ds(0, sz)], out_ref.at[pl.ds(lo, sz), :],
            sem_out.at[c % _NBUF])

    for step in range(nc + _LAG):
        if step < nc:
            if step >= _NBUF:
                out_cp(step - _NBUF).wait()
            in_cp(step).start()
        c_proc = step - _LAG
        if c_proc >= 0:
            in_cp(c_proc).wait()
            out_cp(c_proc).start()

    attr_cp.wait()
    for cp in tail_cps:
        cp.wait()
    _compute_rows(tails_vmem, attr_vmem)

    tail_out = []
    for i in range(2):
        cp = pltpu.make_async_copy(
            tails_vmem.at[i],
            out_ref.at[pl.ds((i + 1) * half - _TAIL, _TAIL), :],
            sem_tail.at[i])
        cp.start()
        tail_out.append(cp)

    for c in range(nc - _NBUF, nc):
        out_cp(c).wait()
    for cp in tail_out:
        cp.wait()


def kernel(hidden_states, attractors):
    b, s, d = hidden_states.shape
    k = attractors.shape[0]
    flat = hidden_states.reshape(b * s, d)
    out = pl.pallas_call(
        _body,
        in_specs=[
            pl.BlockSpec(memory_space=pltpu.HBM),
            pl.BlockSpec(memory_space=pltpu.HBM),
        ],
        out_specs=pl.BlockSpec(memory_space=pltpu.HBM),
        out_shape=jax.ShapeDtypeStruct((b * s, d), hidden_states.dtype),
        scratch_shapes=[
            pltpu.VMEM((_NBUF, _CH, d), jnp.float32),
            pltpu.VMEM((k, d), jnp.float32),
            pltpu.VMEM((2, _TAIL, d), jnp.float32),
            pltpu.SemaphoreType.DMA((_NBUF,)),
            pltpu.SemaphoreType.DMA((_NBUF,)),
            pltpu.SemaphoreType.DMA,
            pltpu.SemaphoreType.DMA((2,)),
        ],
    )(flat, attractors)
    return out.reshape(b, s, d)
